# baseline (speedup 1.0000x reference)
import numpy as np
import jax
import jax.numpy as jnp
from jax.experimental import pallas as pl
from jax.experimental.pallas import tpu as pltpu

_N_POINTS = 1048576
_N_IN = 64
_N_OUT = 32
_IDX = np.concatenate([4 * l + np.arange(2) for l in range(16)]).astype(np.int32)

_BLOCK = 4096


def _body(h_ref, o_ref):
    x = h_ref[...]
    parts = [x[:, 4 * l:4 * l + 2] for l in range(16)]
    o_ref[...] = jnp.concatenate(parts, axis=1)


def kernel(h):
    n = h.shape[0]
    grid = n // _BLOCK
    return pl.pallas_call(
        _body,
        grid=(grid,),
        in_specs=[pl.BlockSpec((_BLOCK, _N_IN), lambda i: (i, 0))],
        out_specs=pl.BlockSpec((_BLOCK, _N_OUT), lambda i: (i, 0)),
        out_shape=jax.ShapeDtypeStruct((n, _N_OUT), h.dtype),
    )(h)
